# baseline (speedup 1.0000x reference)
import jax
import jax.numpy as jnp
from jax import lax
from jax.experimental import pallas as pl
from jax.experimental.pallas import tpu as pltpu
from jax.experimental.pallas import tpu_sc as plsc

_B, _S, _D, _V, _P, _K = 32, 8, 4096, 100000, 512, 20
_ROWS = _B * _S
_NC, _NS, _L = 2, 16, 16
_NW = _NC * _NS
_RPW = _ROWS // _NW
_NVREG = _V // _L
_GRP = 8
_NGRP = 782
_VPAD = 784 * _GRP * _L
_GMAX = _NGRP * _L
_CAP = 64
_CBUF = _CAP + _L

_NEG = float("-inf")
_BIG = 2**30

_CG0 = (0, 194, 388, 582, 776, 781)
_CGN = (194, 194, 194, 194, 5, 1)
_CW0 = (0, 24832, 49664, 74496, 99328, 99968)
_CWN = (24832, 24832, 24832, 24832, 640, 32)
_NCH = 6


def _flags_body(h_ref, p_ref, o_ref):
    emb = jnp.mean(h_ref[...], axis=1)
    protos = p_ref[...]
    na = jnp.sqrt(jnp.sum(emb * emb, axis=1, keepdims=True))
    ones = jnp.ones((1, _D), dtype=jnp.float32)
    nb_sq = lax.dot_general(
        ones, protos * protos, (((1,), (1,)), ((), ())),
        preferred_element_type=jnp.float32,
        precision=lax.Precision.HIGHEST)
    nb = jnp.sqrt(nb_sq)
    dots = lax.dot_general(
        emb, protos, (((1,), (1,)), ((), ())),
        preferred_element_type=jnp.float32,
        precision=lax.Precision.HIGHEST)
    sims = dots / jnp.maximum(na * nb, 1e-8)
    sim = jnp.mean(sims, axis=1, keepdims=True)
    o_ref[...] = (sim > 0.0).astype(jnp.float32)[:, :, None]


def _sc_body(x_hbm, xt_hbm, f_hbm, o_hbm, ot_hbm, row_v, gmax_v, cval_v,
             cidx_v, flg_v, idx_v, in_sem, out_sem, patch_sem):
    wid = lax.axis_index("s") * _NC + lax.axis_index("c")
    pltpu.sync_copy(f_hbm, flg_v)
    lanes = lax.iota(jnp.int32, _L)
    f0 = flg_v[pl.ds(0, _L)]
    f1 = flg_v[pl.ds(_L, _L)]
    zf = jnp.full((_L,), 0.0, jnp.float32)
    fsel = (jnp.where(lanes == wid, f0, zf)
            + jnp.where(lanes == wid - _L, f1, zf))
    suppress = plsc.all_reduce_population_count(fsel > 0.5)[0] > 0

    def _bfly(v, op):
        for sh in (8, 4, 2, 1):
            v = op(v, v.at[lanes ^ sh].get(mode="promise_in_bounds"))
        return v

    for t in range(_NVREG, 784 * _GRP):
        row_v[pl.ds(t * _L, _L)] = jnp.full((_L,), _NEG)

    def _inc(row, c):
        if c == _NCH - 1:
            return pltpu.make_async_copy(
                xt_hbm.at[pl.ds(row * 32, 32)],
                row_v.at[pl.ds(_CW0[c], _CWN[c])], in_sem)
        return pltpu.make_async_copy(
            x_hbm.at[row, pl.ds(_CW0[c], _CWN[c])],
            row_v.at[pl.ds(_CW0[c], _CWN[c])], in_sem)

    def _outc(row, c):
        if c == _NCH - 1:
            return pltpu.make_async_copy(
                row_v.at[pl.ds(_CW0[c], _CWN[c])],
                ot_hbm.at[pl.ds(row * 32, 32)], out_sem)
        return pltpu.make_async_copy(
            row_v.at[pl.ds(_CW0[c], _CWN[c])],
            o_hbm.at[row, pl.ds(_CW0[c], _CWN[c])], out_sem)

    row0 = wid * _RPW
    for c in range(_NCH):
        _inc(row0, c).start()

    def _patch(row, k):
        iv = idx_v[pl.ds(0, _L)] if k < _L else idx_v[pl.ds(_L, _L)]
        idx = iv[k % _L]

        def descr():
            c0 = pl.multiple_of((idx >> 7) << 7, 128)
            return pltpu.make_async_copy(
                row_v.at[pl.ds(c0, 128)],
                o_hbm.at[row, pl.ds(c0, 128)], patch_sem)

        def descr_tail():
            return pltpu.make_async_copy(
                row_v.at[pl.ds(_V - 32, 32)],
                ot_hbm.at[pl.ds(row * 32, 32)], patch_sem)
        return idx, descr, descr_tail

    def rloop(j, carry):
        base = wid * _RPW + j
        for c in range(_NCH):
            _inc(base, c).wait()
            _outc(base, c).start()

            @pl.when(suppress)
            def _(c=c):
                def p1(g, c1):
                    gb = g * _GRP * _L
                    a0 = jnp.full((_L,), _NEG)
                    a1 = jnp.full((_L,), _NEG)
                    a2 = jnp.full((_L,), _NEG)
                    a3 = jnp.full((_L,), _NEG)
                    for u in range(0, _GRP, 4):
                        a0 = jnp.maximum(a0, row_v[pl.ds(gb + u * _L, _L)])
                        a1 = jnp.maximum(a1, row_v[pl.ds(gb + (u + 1) * _L, _L)])
                        a2 = jnp.maximum(a2, row_v[pl.ds(gb + (u + 2) * _L, _L)])
                        a3 = jnp.maximum(a3, row_v[pl.ds(gb + (u + 3) * _L, _L)])
                    gmax_v[pl.ds(g * _L, _L)] = jnp.maximum(
                        jnp.maximum(a0, a1), jnp.maximum(a2, a3))
                    return c1
                lax.fori_loop(_CG0[c], _CG0[c] + _CGN[c], p1, 0)

        @pl.when(suppress)
        def _():
            def p1b(g, carry2):
                g1, g2 = carry2
                v = gmax_v[pl.ds(g * _L, _L)]
                g2 = jnp.maximum(g2, jnp.minimum(g1, v))
                g1 = jnp.maximum(g1, v)
                return g1, g2
            g1, g2 = lax.fori_loop(
                0, _NGRP, p1b, (jnp.full((_L,), _NEG), jnp.full((_L,), _NEG)))

            def sel(k, carry2):
                a, b2, _ = carry2
                m = _bfly(jnp.maximum(a, b2), jnp.maximum)
                return (jnp.where(a >= m, _NEG, a),
                        jnp.where(b2 >= m, _NEG, b2), m)
            _, _, t_ap = lax.fori_loop(
                0, _K, sel,
                (g1, g2, jnp.full((_L,), 0.0, jnp.float32)))

            for t in range(_CBUF // _L):
                cval_v[pl.ds(t * _L, _L)] = jnp.full((_L,), _NEG)
                cidx_v[pl.ds(t * _L, _L)] = jnp.full((_L,), _BIG)

            def p2(g, cnt):
                hits = plsc.all_reduce_population_count(
                    gmax_v[pl.ds(g * _L, _L)] >= t_ap)

                def scan_grp(cnt_in):
                    for u in range(_GRP):
                        i = g * _GRP + u
                        v = row_v[pl.ds(i * _L, _L)]
                        msk = jnp.logical_and(
                            v >= t_ap, jnp.full((_L,), cnt_in < _CAP))
                        plsc.store_compressed(
                            cval_v.at[pl.ds(cnt_in, _L)], v, mask=msk)
                        plsc.store_compressed(
                            cidx_v.at[pl.ds(cnt_in, _L)], lanes + i * _L,
                            mask=msk)
                        n = plsc.all_reduce_population_count(msk)
                        cnt_in = cnt_in + n[0]
                    return cnt_in
                return lax.cond(hits[0] > 0, scan_grp, lambda cc: cc, cnt)
            lax.fori_loop(0, _NGRP, p2, jnp.int32(0))

            def ext(k, carry2):
                i0, i1 = carry2
                macc = jnp.full((_L,), _NEG)
                for t in range(_CBUF // _L):
                    macc = jnp.maximum(macc, cval_v[pl.ds(t * _L, _L)])
                m = _bfly(macc, jnp.maximum)
                iacc = jnp.full((_L,), _BIG)
                for t in range(_CBUF // _L):
                    cv = cval_v[pl.ds(t * _L, _L)]
                    ci = cidx_v[pl.ds(t * _L, _L)]
                    iacc = jnp.minimum(iacc, jnp.where(cv == m, ci, _BIG))
                isel = _bfly(iacc, jnp.minimum)
                for t in range(_CBUF // _L):
                    cv = cval_v[pl.ds(t * _L, _L)]
                    ci = cidx_v[pl.ds(t * _L, _L)]
                    hit = jnp.logical_and(cv == m, ci == isel)
                    cval_v[pl.ds(t * _L, _L)] = jnp.where(hit, _NEG, cv)
                i0 = jnp.where(lanes == k, isel, i0)
                i1 = jnp.where(lanes == k - _L, isel, i1)
                return i0, i1
            zeros = jnp.full((_L,), jnp.int32(0))
            i0, i1 = lax.fori_loop(0, _K, ext, (zeros, zeros))
            idx_v[pl.ds(0, _L)] = i0
            idx_v[pl.ds(_L, _L)] = jnp.where(lanes < _K - _L, i1, _BIG)

        for c in range(_NCH):
            _outc(base, c).wait()

        @pl.when(suppress)
        def _():
            i0 = idx_v[pl.ds(0, _L)]
            i1 = idx_v[pl.ds(_L, _L)]
            neg100 = jnp.full((_L,), jnp.float32(-100.0))
            plsc.store_scatter(row_v, [i0], neg100, mask=i0 < jnp.int32(_V))
            plsc.store_scatter(row_v, [i1], neg100, mask=i1 < jnp.int32(_V))
            for k in range(_K):
                idx, descr, descr_tail = _patch(base, k)

                @pl.when(jnp.logical_and(idx < _V, idx < _V - 32))
                def _(descr=descr):
                    descr().start()

                @pl.when(jnp.logical_and(idx < _V, idx >= _V - 32))
                def _(descr_tail=descr_tail):
                    descr_tail().start()
            for k in range(_K):
                idx, descr, descr_tail = _patch(base, k)

                @pl.when(jnp.logical_and(idx < _V, idx < _V - 32))
                def _(descr=descr):
                    descr().wait()

                @pl.when(jnp.logical_and(idx < _V, idx >= _V - 32))
                def _(descr_tail=descr_tail):
                    descr_tail().wait()

        @pl.when(j < _RPW - 1)
        def _():
            for c in range(_NCH):
                _inc(base + 1, c).start()
        return carry

    lax.fori_loop(0, _RPW, rloop, 0)


def kernel(hidden_states, logits, prototypes):
    flags = pl.pallas_call(
        _flags_body,
        out_shape=jax.ShapeDtypeStruct((_B, 1, 1), jnp.float32),
    )(hidden_states, prototypes)

    sc = pl.kernel(
        _sc_body,
        out_type=(jax.ShapeDtypeStruct((_ROWS, _V), jnp.float32),
                  jax.ShapeDtypeStruct((_ROWS * 32,), jnp.float32)),
        mesh=plsc.VectorSubcoreMesh(core_axis_name="c", subcore_axis_name="s"),
        compiler_params=pltpu.CompilerParams(needs_layout_passes=False),
        scratch_types=[
            pltpu.VMEM((_VPAD,), jnp.float32),
            pltpu.VMEM((_GMAX,), jnp.float32),
            pltpu.VMEM((_CBUF,), jnp.float32),
            pltpu.VMEM((_CBUF,), jnp.int32),
            pltpu.VMEM((_B,), jnp.float32),
            pltpu.VMEM((2 * _L,), jnp.int32),
            pltpu.SemaphoreType.DMA,
            pltpu.SemaphoreType.DMA,
            pltpu.SemaphoreType.DMA,
        ],
    )
    x2 = logits.reshape(_ROWS, _V)
    x_tail = x2[:, _V - 32:].reshape(_ROWS * 32)
    out, out_tail = sc(x2, x_tail, flags.reshape(_B))
    out = out.at[:, _V - 32:].set(out_tail.reshape(_ROWS, 32))
    return out.reshape(_B, 1, _S, _V)


# speedup vs baseline: 1.3818x; 1.3818x over previous
import jax
import jax.numpy as jnp
from jax import lax
from jax.experimental import pallas as pl
from jax.experimental.pallas import tpu as pltpu
from jax.experimental.pallas import tpu_sc as plsc

_B, _S, _D, _V, _P, _K = 32, 8, 4096, 100000, 512, 20
_ROWS = _B * _S
_NC, _NS, _L = 2, 16, 16
_NW = _NC * _NS
_RPW = _ROWS // _NW
_NVREG = _V // _L
_GRP = 8
_NGRP = 782
_VPAD = 784 * _GRP * _L
_GMAX = 784 * _L
_NSG = 49
_CAP = 64
_CBUF = _CAP + _L

_NEG = float("-inf")
_BIG = 2**30

_CG0 = (0, 194, 388, 582, 776, 781)
_CGN = (194, 194, 194, 194, 5, 1)
_CW0 = (0, 24832, 49664, 74496, 99328, 99968)
_CWN = (24832, 24832, 24832, 24832, 640, 32)
_NCH = 6


def _flags_body(h_ref, p_ref, o_ref):
    emb = jnp.mean(h_ref[...], axis=1)
    protos = p_ref[...]
    na = jnp.sqrt(jnp.sum(emb * emb, axis=1, keepdims=True))
    ones = jnp.ones((1, _D), dtype=jnp.float32)
    nb_sq = lax.dot_general(
        ones, protos * protos, (((1,), (1,)), ((), ())),
        preferred_element_type=jnp.float32,
        precision=lax.Precision.HIGHEST)
    nb = jnp.sqrt(nb_sq)
    dots = lax.dot_general(
        emb, protos, (((1,), (1,)), ((), ())),
        preferred_element_type=jnp.float32,
        precision=lax.Precision.HIGHEST)
    sims = dots / jnp.maximum(na * nb, 1e-8)
    sim = jnp.mean(sims, axis=1, keepdims=True)
    o_ref[...] = (sim > 0.0).astype(jnp.float32)[:, :, None]


def _sc_body(x_hbm, xt_hbm, f_hbm, o_hbm, ot_hbm, row_v, gmax_v, sgmax_v,
             cval_v,
             cidx_v, flg_v, idx_v, in_sem, out_sem, patch_sem):
    wid = lax.axis_index("s") * _NC + lax.axis_index("c")
    pltpu.sync_copy(f_hbm, flg_v)
    lanes = lax.iota(jnp.int32, _L)
    f0 = flg_v[pl.ds(0, _L)]
    f1 = flg_v[pl.ds(_L, _L)]
    zf = jnp.full((_L,), 0.0, jnp.float32)
    fsel = (jnp.where(lanes == wid, f0, zf)
            + jnp.where(lanes == wid - _L, f1, zf))
    suppress = plsc.all_reduce_population_count(fsel > 0.5)[0] > 0

    def _bfly(v, op):
        for sh in (8, 4, 2, 1):
            v = op(v, v.at[lanes ^ sh].get(mode="promise_in_bounds"))
        return v

    for t in range(_NVREG, 784 * _GRP):
        row_v[pl.ds(t * _L, _L)] = jnp.full((_L,), _NEG)
    for g in (782, 783):
        gmax_v[pl.ds(g * _L, _L)] = jnp.full((_L,), _NEG)

    def _inc(row, c):
        if c == _NCH - 1:
            return pltpu.make_async_copy(
                xt_hbm.at[pl.ds(row * 32, 32)],
                row_v.at[pl.ds(_CW0[c], _CWN[c])], in_sem)
        return pltpu.make_async_copy(
            x_hbm.at[row, pl.ds(_CW0[c], _CWN[c])],
            row_v.at[pl.ds(_CW0[c], _CWN[c])], in_sem)

    def _outc(row, c):
        if c == _NCH - 1:
            return pltpu.make_async_copy(
                row_v.at[pl.ds(_CW0[c], _CWN[c])],
                ot_hbm.at[pl.ds(row * 32, 32)], out_sem)
        return pltpu.make_async_copy(
            row_v.at[pl.ds(_CW0[c], _CWN[c])],
            o_hbm.at[row, pl.ds(_CW0[c], _CWN[c])], out_sem)

    row0 = wid * _RPW
    for c in range(_NCH):
        _inc(row0, c).start()

    def _patch(row, k):
        iv = idx_v[pl.ds(0, _L)] if k < _L else idx_v[pl.ds(_L, _L)]
        idx = iv[k % _L]

        def descr():
            c0 = pl.multiple_of((idx >> 7) << 7, 128)
            return pltpu.make_async_copy(
                row_v.at[pl.ds(c0, 128)],
                o_hbm.at[row, pl.ds(c0, 128)], patch_sem)

        def descr_tail():
            return pltpu.make_async_copy(
                row_v.at[pl.ds(_V - 32, 32)],
                ot_hbm.at[pl.ds(row * 32, 32)], patch_sem)
        return idx, descr, descr_tail

    def rloop(j, carry):
        base = wid * _RPW + j
        for c in range(_NCH):
            _inc(base, c).wait()
            _outc(base, c).start()

            @pl.when(suppress)
            def _(c=c):
                def p1(g, c1):
                    gb = g * _GRP * _L
                    a0 = jnp.full((_L,), _NEG)
                    a1 = jnp.full((_L,), _NEG)
                    a2 = jnp.full((_L,), _NEG)
                    a3 = jnp.full((_L,), _NEG)
                    for u in range(0, _GRP, 4):
                        a0 = jnp.maximum(a0, row_v[pl.ds(gb + u * _L, _L)])
                        a1 = jnp.maximum(a1, row_v[pl.ds(gb + (u + 1) * _L, _L)])
                        a2 = jnp.maximum(a2, row_v[pl.ds(gb + (u + 2) * _L, _L)])
                        a3 = jnp.maximum(a3, row_v[pl.ds(gb + (u + 3) * _L, _L)])
                    gmax_v[pl.ds(g * _L, _L)] = jnp.maximum(
                        jnp.maximum(a0, a1), jnp.maximum(a2, a3))
                    return c1
                lax.fori_loop(_CG0[c], _CG0[c] + _CGN[c], p1, 0)

        @pl.when(suppress)
        def _():
            def p1b(sg, carry2):
                g1a, g2a, g1b, g2b = carry2
                sgm = jnp.full((_L,), _NEG)
                for u in range(0, _L, 2):
                    va = gmax_v[pl.ds((sg * _L + u) * _L, _L)]
                    vb = gmax_v[pl.ds((sg * _L + u + 1) * _L, _L)]
                    sgm = jnp.maximum(sgm, jnp.maximum(va, vb))
                    g2a = jnp.maximum(g2a, jnp.minimum(g1a, va))
                    g1a = jnp.maximum(g1a, va)
                    g2b = jnp.maximum(g2b, jnp.minimum(g1b, vb))
                    g1b = jnp.maximum(g1b, vb)
                sgmax_v[pl.ds(sg * _L, _L)] = sgm
                return g1a, g2a, g1b, g2b
            nf = jnp.full((_L,), _NEG)
            g1a, g2a, g1b, g2b = lax.fori_loop(
                0, _NSG, p1b, (nf, nf, nf, nf))
            g1 = jnp.maximum(g1a, g1b)
            g2 = jnp.maximum(jnp.minimum(g1a, g1b), jnp.maximum(g2a, g2b))

            def sel(k, carry2):
                a, b2, _ = carry2
                m = _bfly(jnp.maximum(a, b2), jnp.maximum)
                return (jnp.where(a >= m, _NEG, a),
                        jnp.where(b2 >= m, _NEG, b2), m)
            _, _, t_ap = lax.fori_loop(
                0, _K, sel,
                (g1, g2, jnp.full((_L,), 0.0, jnp.float32)))

            for t in range(_CBUF // _L):
                cval_v[pl.ds(t * _L, _L)] = jnp.full((_L,), _NEG)
                cidx_v[pl.ds(t * _L, _L)] = jnp.full((_L,), _BIG)

            def p2(g, cnt):
                hits = plsc.all_reduce_population_count(
                    gmax_v[pl.ds(g * _L, _L)] >= t_ap)

                def scan_grp(cnt_in):
                    for u in range(_GRP):
                        i = g * _GRP + u
                        v = row_v[pl.ds(i * _L, _L)]
                        msk = jnp.logical_and(
                            v >= t_ap, jnp.full((_L,), cnt_in < _CAP))
                        plsc.store_compressed(
                            cval_v.at[pl.ds(cnt_in, _L)], v, mask=msk)
                        plsc.store_compressed(
                            cidx_v.at[pl.ds(cnt_in, _L)], lanes + i * _L,
                            mask=msk)
                        n = plsc.all_reduce_population_count(msk)
                        cnt_in = cnt_in + n[0]
                    return cnt_in
                return lax.cond(hits[0] > 0, scan_grp, lambda cc: cc, cnt)

            def p2s(sg, cnt):
                sh = plsc.all_reduce_population_count(
                    sgmax_v[pl.ds(sg * _L, _L)] >= t_ap)
                return lax.cond(
                    sh[0] > 0,
                    lambda cc: lax.fori_loop(sg * _L, (sg + 1) * _L, p2, cc),
                    lambda cc: cc, cnt)
            lax.fori_loop(0, _NSG, p2s, jnp.int32(0))

            def ext(k, carry2):
                i0, i1 = carry2
                macc = jnp.full((_L,), _NEG)
                for t in range(_CBUF // _L):
                    macc = jnp.maximum(macc, cval_v[pl.ds(t * _L, _L)])
                m = _bfly(macc, jnp.maximum)
                iacc = jnp.full((_L,), _BIG)
                for t in range(_CBUF // _L):
                    cv = cval_v[pl.ds(t * _L, _L)]
                    ci = cidx_v[pl.ds(t * _L, _L)]
                    iacc = jnp.minimum(iacc, jnp.where(cv == m, ci, _BIG))
                isel = _bfly(iacc, jnp.minimum)
                for t in range(_CBUF // _L):
                    cv = cval_v[pl.ds(t * _L, _L)]
                    ci = cidx_v[pl.ds(t * _L, _L)]
                    hit = jnp.logical_and(cv == m, ci == isel)
                    cval_v[pl.ds(t * _L, _L)] = jnp.where(hit, _NEG, cv)
                i0 = jnp.where(lanes == k, isel, i0)
                i1 = jnp.where(lanes == k - _L, isel, i1)
                return i0, i1
            zeros = jnp.full((_L,), jnp.int32(0))
            i0, i1 = lax.fori_loop(0, _K, ext, (zeros, zeros))
            idx_v[pl.ds(0, _L)] = i0
            idx_v[pl.ds(_L, _L)] = jnp.where(lanes < _K - _L, i1, _BIG)

        for c in range(_NCH):
            _outc(base, c).wait()

        @pl.when(suppress)
        def _():
            i0 = idx_v[pl.ds(0, _L)]
            i1 = idx_v[pl.ds(_L, _L)]
            neg100 = jnp.full((_L,), jnp.float32(-100.0))
            plsc.store_scatter(row_v, [i0], neg100, mask=i0 < jnp.int32(_V))
            plsc.store_scatter(row_v, [i1], neg100, mask=i1 < jnp.int32(_V))
            for k in range(_K):
                idx, descr, descr_tail = _patch(base, k)

                @pl.when(jnp.logical_and(idx < _V, idx < _V - 32))
                def _(descr=descr):
                    descr().start()

                @pl.when(jnp.logical_and(idx < _V, idx >= _V - 32))
                def _(descr_tail=descr_tail):
                    descr_tail().start()
            for k in range(_K):
                idx, descr, descr_tail = _patch(base, k)

                @pl.when(jnp.logical_and(idx < _V, idx < _V - 32))
                def _(descr=descr):
                    descr().wait()

                @pl.when(jnp.logical_and(idx < _V, idx >= _V - 32))
                def _(descr_tail=descr_tail):
                    descr_tail().wait()

        @pl.when(j < _RPW - 1)
        def _():
            for c in range(_NCH):
                _inc(base + 1, c).start()
        return carry

    lax.fori_loop(0, _RPW, rloop, 0)


def kernel(hidden_states, logits, prototypes):
    flags = pl.pallas_call(
        _flags_body,
        out_shape=jax.ShapeDtypeStruct((_B, 1, 1), jnp.float32),
    )(hidden_states, prototypes)

    sc = pl.kernel(
        _sc_body,
        out_type=(jax.ShapeDtypeStruct((_ROWS, _V), jnp.float32),
                  jax.ShapeDtypeStruct((_ROWS * 32,), jnp.float32)),
        mesh=plsc.VectorSubcoreMesh(core_axis_name="c", subcore_axis_name="s"),
        compiler_params=pltpu.CompilerParams(needs_layout_passes=False),
        scratch_types=[
            pltpu.VMEM((_VPAD,), jnp.float32),
            pltpu.VMEM((_GMAX,), jnp.float32),
            pltpu.VMEM((_NSG * _L,), jnp.float32),
            pltpu.VMEM((_CBUF,), jnp.float32),
            pltpu.VMEM((_CBUF,), jnp.int32),
            pltpu.VMEM((_B,), jnp.float32),
            pltpu.VMEM((2 * _L,), jnp.int32),
            pltpu.SemaphoreType.DMA,
            pltpu.SemaphoreType.DMA,
            pltpu.SemaphoreType.DMA,
        ],
    )
    x2 = logits.reshape(_ROWS, _V)
    x_tail = x2[:, _V - 32:].reshape(_ROWS * 32)
    out, out_tail = sc(x2, x_tail, flags.reshape(_B))
    out = out.at[:, _V - 32:].set(out_tail.reshape(_ROWS, 32))
    return out.reshape(_B, 1, _S, _V)
